# baseline (speedup 1.0000x reference)
import functools

import jax
import jax.numpy as jnp
import numpy as np
from jax.experimental import pallas as pl
from jax.experimental.pallas import tpu as pltpu

_N = 20000
_C = 81
_IMW = 1024.0
_IMH = 1024.0
_CLIP = float(np.log(1000.0 / 16.0))
_R2D = float(180.0 / np.pi)


def _body(logits_ref, codes_ref, props_ref, boxes_ref, scores_ref):
    logits = logits_ref[...]
    m = jnp.max(logits, axis=-1, keepdims=True)
    p = jnp.exp(logits - m)
    s = jnp.sum(p, axis=-1, keepdims=True)
    scores_ref[...] = p / s

    codes = codes_ref[...]
    props = props_ref[...]
    cx = props[:, 0:1]
    cy = props[:, 1:2]
    w = props[:, 2:3]
    h = props[:, 3:4]
    a = props[:, 4:5]

    r, ncol = codes.shape
    t = jax.lax.broadcasted_iota(jnp.int32, (r, ncol), 1) % 5
    is_xy = t < 2
    is_wh = (t == 2) | (t == 3)
    use_w = (t == 0) | (t == 2)
    use_h = (t == 1) | (t == 3)

    scale = jnp.where(is_xy, 0.1, jnp.where(is_wh, 0.2, 1.0))
    d = codes * scale
    e = jnp.exp(jnp.minimum(d, _CLIP))
    base = jnp.where(is_wh, e, d)
    mult = jnp.where(use_w, w, jnp.where(use_h, h, _R2D))
    addv = jnp.where(t == 0, cx, jnp.where(t == 1, cy, jnp.where(t == 4, a, 0.0)))
    out = base * mult + addv
    out = jnp.where(is_xy, jnp.clip(out, 0.0, _IMW - 1.0), out)
    boxes_ref[...] = out


@functools.partial(jax.jit, static_argnums=(3,))
def _run(class_logits, box_regression, proposals, block_rows):
    n = class_logits.shape[0]
    grid = (n // block_rows,)
    boxes, scores = pl.pallas_call(
        _body,
        grid=grid,
        in_specs=[
            pl.BlockSpec((block_rows, _C), lambda i: (i, 0)),
            pl.BlockSpec((block_rows, _C * 5), lambda i: (i, 0)),
            pl.BlockSpec((block_rows, 5), lambda i: (i, 0)),
        ],
        out_specs=[
            pl.BlockSpec((block_rows, _C * 5), lambda i: (i, 0)),
            pl.BlockSpec((block_rows, _C), lambda i: (i, 0)),
        ],
        out_shape=[
            jax.ShapeDtypeStruct((n, _C * 5), jnp.float32),
            jax.ShapeDtypeStruct((n, _C), jnp.float32),
        ],
        compiler_params=pltpu.CompilerParams(
            dimension_semantics=("parallel",),
        ),
    )(class_logits, box_regression, proposals)
    return boxes.reshape(-1, 5), scores.reshape(-1)


def kernel(class_logits, box_regression, proposals, num_of_fwd_left=0):
    return _run(class_logits, box_regression, proposals, 1000)
